# Initial kernel scaffold; baseline (speedup 1.0000x reference)
import jax
import jax.numpy as jnp
from jax.experimental import pallas as pl


def kernel(table, user_ids, item_ids):
    raise NotImplementedError("write your pallas kernel here")


# speedup vs baseline: 139.3485x; 139.3485x over previous
import functools

import jax
import jax.numpy as jnp
from jax import lax
from jax.experimental import pallas as pl
from jax.experimental.pallas import tpu as pltpu
from jax.experimental.pallas import tpu_sc as plsc

_NUM_CORES = 2
_NUM_SUBCORES = 16
_NUM_WORKERS = _NUM_CORES * _NUM_SUBCORES


@functools.lru_cache(maxsize=None)
def _build_gather(n_total: int, chunk: int):
    per_w = n_total // _NUM_WORKERS
    n_chunks = per_w // chunk
    mesh = plsc.VectorSubcoreMesh(core_axis_name="c", subcore_axis_name="s")

    @functools.partial(
        pl.kernel,
        out_type=jax.ShapeDtypeStruct((n_total,), jnp.float32),
        mesh=mesh,
        scratch_types=[
            pltpu.VMEM((chunk,), jnp.int32),
            pltpu.VMEM((chunk,), jnp.float32),
            pltpu.SemaphoreType.DMA,
        ],
    )
    def gather(table_hbm, idx_hbm, out_hbm, idx_v, rows_v, sem):
        wid = lax.axis_index("s") * _NUM_CORES + lax.axis_index("c")
        base = wid * per_w
        for i in range(n_chunks):
            off = base + i * chunk
            pltpu.sync_copy(idx_hbm.at[pl.ds(off, chunk)], idx_v)
            pltpu.async_copy(table_hbm.at[idx_v], rows_v, sem).wait()
            pltpu.sync_copy(rows_v, out_hbm.at[pl.ds(off, chunk)])

    return gather


def kernel(table, user_ids, item_ids):
    del user_ids
    b, h = item_ids.shape
    n_total = b * h
    idx = item_ids.reshape(n_total).astype(jnp.int32)
    out = _build_gather(n_total, 25600)(table, idx)
    return out.reshape(b, h)


# speedup vs baseline: 216.0391x; 1.5504x over previous
import functools

import jax
import jax.numpy as jnp
from jax import lax
from jax.experimental import pallas as pl
from jax.experimental.pallas import tpu as pltpu
from jax.experimental.pallas import tpu_sc as plsc

_NUM_CORES = 2
_NUM_SUBCORES = 16
_NUM_WORKERS = _NUM_CORES * _NUM_SUBCORES


@functools.lru_cache(maxsize=None)
def _build_gather(n_total: int, chunk: int):
    per_w = n_total // _NUM_WORKERS
    n_chunks = per_w // chunk
    mesh = plsc.VectorSubcoreMesh(core_axis_name="c", subcore_axis_name="s")

    vocab = 1000000
    n_full = vocab // chunk
    tail = vocab - n_full * chunk
    n_pieces = n_full + (1 if tail else 0)
    assert chunk % 8 == 0 and tail % 8 == 0
    rounds = -(-n_pieces // _NUM_SUBCORES)

    @functools.partial(
        pl.kernel,
        out_type=jax.ShapeDtypeStruct((n_total,), jnp.float32),
        mesh=mesh,
        scratch_types=[
            pltpu.VMEM((chunk,), jnp.int32),
            pltpu.VMEM((chunk,), jnp.float32),
            pltpu.VMEM_SHARED((vocab,), jnp.float32),
            pltpu.SemaphoreType.DMA,
        ],
    )
    def gather(table_hbm, idx_hbm, out_hbm, idx_v, rows_v, sh_table, sem):
        sid = lax.axis_index("s")
        wid = sid * _NUM_CORES + lax.axis_index("c")
        base = wid * per_w
        for r in range(rounds):
            p = sid + r * _NUM_SUBCORES

            @pl.when(p < n_full)
            def _():
                s0 = p * chunk
                pltpu.sync_copy(table_hbm.at[pl.ds(s0, chunk)], rows_v)
                pltpu.sync_copy(rows_v, sh_table.at[pl.ds(s0, chunk)])

            if tail:

                @pl.when(p == n_full)
                def _():
                    t0 = n_full * chunk
                    pltpu.sync_copy(
                        table_hbm.at[pl.ds(t0, tail)], rows_v.at[pl.ds(0, tail)]
                    )
                    pltpu.sync_copy(
                        rows_v.at[pl.ds(0, tail)], sh_table.at[pl.ds(t0, tail)]
                    )

        plsc.subcore_barrier()
        for i in range(n_chunks):
            off = base + i * chunk
            pltpu.sync_copy(idx_hbm.at[pl.ds(off, chunk)], idx_v)
            pltpu.async_copy(sh_table.at[idx_v], rows_v, sem).wait()
            pltpu.sync_copy(rows_v, out_hbm.at[pl.ds(off, chunk)])

    return gather


def kernel(table, user_ids, item_ids):
    del user_ids
    b, h = item_ids.shape
    n_total = b * h
    idx = item_ids.reshape(n_total).astype(jnp.int32)
    out = _build_gather(n_total, 25600)(table, idx)
    return out.reshape(b, h)


# speedup vs baseline: 232.9489x; 1.0783x over previous
import functools

import jax
import jax.numpy as jnp
from jax import lax
from jax.experimental import pallas as pl
from jax.experimental.pallas import tpu as pltpu
from jax.experimental.pallas import tpu_sc as plsc

_NUM_CORES = 2
_NUM_SUBCORES = 16
_NUM_WORKERS = _NUM_CORES * _NUM_SUBCORES


@functools.lru_cache(maxsize=None)
def _build_gather(n_total: int, vocab: int, chunk: int):
    per_w = n_total // _NUM_WORKERS
    n_chunks = per_w // chunk
    assert n_chunks >= 2 and per_w % chunk == 0
    mesh = plsc.VectorSubcoreMesh(core_axis_name="c", subcore_axis_name="s")

    piece = chunk
    n_full = vocab // piece
    tail = vocab - n_full * piece
    n_pieces = n_full + (1 if tail else 0)
    assert piece % 8 == 0 and tail % 8 == 0
    rounds = -(-n_pieces // _NUM_SUBCORES)

    @functools.partial(
        pl.kernel,
        out_type=jax.ShapeDtypeStruct((n_total,), jnp.float32),
        mesh=mesh,
        scratch_types=[
            pltpu.VMEM((chunk,), jnp.int32),
            pltpu.VMEM((chunk,), jnp.int32),
            pltpu.VMEM((chunk,), jnp.float32),
            pltpu.VMEM((chunk,), jnp.float32),
            pltpu.VMEM_SHARED((vocab,), jnp.float32),
            pltpu.SemaphoreType.DMA,
            pltpu.SemaphoreType.DMA,
            pltpu.SemaphoreType.DMA,
            pltpu.SemaphoreType.DMA,
            pltpu.SemaphoreType.DMA,
        ],
    )
    def gather(
        table_hbm,
        idx_hbm,
        out_hbm,
        idx0,
        idx1,
        rows0,
        rows1,
        sh_table,
        sem_in0,
        sem_in1,
        sem_out0,
        sem_out1,
        sem_g,
    ):
        sid = lax.axis_index("s")
        wid = sid * _NUM_CORES + lax.axis_index("c")
        base = wid * per_w
        idx_b = (idx0, idx1)
        rows_b = (rows0, rows1)
        sem_in = (sem_in0, sem_in1)
        sem_out = (sem_out0, sem_out1)

        def in_copy(i):
            b = i % 2
            return pltpu.make_async_copy(
                idx_hbm.at[pl.ds(base + i * chunk, chunk)], idx_b[b], sem_in[b]
            )

        def out_copy(i):
            b = i % 2
            return pltpu.make_async_copy(
                rows_b[b], out_hbm.at[pl.ds(base + i * chunk, chunk)], sem_out[b]
            )

        in_copy(0).start()
        in_copy(1).start()

        def stage_load(r):
            p = sid + r * _NUM_SUBCORES
            b = r % 2

            @pl.when(p < n_full)
            def _():
                pltpu.make_async_copy(
                    table_hbm.at[pl.ds(p * piece, piece)], rows_b[b], sem_out[b]
                ).start()

            if tail:

                @pl.when(p == n_full)
                def _():
                    pltpu.make_async_copy(
                        table_hbm.at[pl.ds(n_full * piece, tail)],
                        rows_b[b].at[pl.ds(0, tail)],
                        sem_out[b],
                    ).start()

        def stage_store(r):
            p = sid + r * _NUM_SUBCORES
            b = r % 2

            @pl.when(p < n_full)
            def _():
                pltpu.make_async_copy(
                    table_hbm.at[pl.ds(p * piece, piece)], rows_b[b], sem_out[b]
                ).wait()
                pltpu.sync_copy(rows_b[b], sh_table.at[pl.ds(p * piece, piece)])

            if tail:

                @pl.when(p == n_full)
                def _():
                    pltpu.make_async_copy(
                        table_hbm.at[pl.ds(n_full * piece, tail)],
                        rows_b[b].at[pl.ds(0, tail)],
                        sem_out[b],
                    ).wait()
                    pltpu.sync_copy(
                        rows_b[b].at[pl.ds(0, tail)],
                        sh_table.at[pl.ds(n_full * piece, tail)],
                    )

        stage_load(0)
        for r in range(rounds):
            if r + 1 < rounds:
                stage_load(r + 1)
            stage_store(r)

        plsc.subcore_barrier()

        for i in range(n_chunks):
            b = i % 2
            in_copy(i).wait()
            if i >= 2:
                out_copy(i - 2).wait()
            pltpu.async_copy(sh_table.at[idx_b[b]], rows_b[b], sem_g).wait()
            out_copy(i).start()
            if i + 2 < n_chunks:
                in_copy(i + 2).start()
        out_copy(n_chunks - 2).wait()
        out_copy(n_chunks - 1).wait()

    return gather


def kernel(table, user_ids, item_ids):
    del user_ids
    b, h = item_ids.shape
    n_total = b * h
    idx = item_ids.reshape(n_total)
    out = _build_gather(n_total, table.shape[0], 12800)(table, idx)
    return out.reshape(b, h)
